# baseline (device time: 47352 ns/iter reference)
import jax
import jax.numpy as jnp
from jax import lax
from jax.experimental import pallas as pl
from jax.experimental.pallas import tpu as pltpu

N_DEV = 4
CAP = 204.0


def kernel(x, router_W, route_idx, expert_W):
    n_tok, d_model = x.shape
    e_local, _, d_ff = expert_W.shape
    n_exp = N_DEV * e_local

    route = route_idx[:, :1].astype(jnp.int32)
    onehot = (route == jnp.arange(n_exp, dtype=jnp.int32)[None, :]).astype(
        jnp.float32
    )
    excl_tok = ((jnp.cumsum(onehot, axis=0) - onehot) * onehot).sum(
        axis=1, keepdims=True
    )
    counts = onehot.sum(axis=0, keepdims=True)

    def body(
        x_ref, route_ref, excl_ref, cnt_ref, ew_ref, out_ref,
        comm, cnt_recv, wsend, wrecv, csend, crecv,
    ):
        my = lax.axis_index("i")
        right = lax.rem(my + 1, N_DEV)

        barrier = pltpu.get_barrier_semaphore()
        for d in range(1, N_DEV):
            peer = lax.rem(my + d, N_DEV)
            pl.semaphore_signal(
                barrier, inc=1, device_id=(peer,),
                device_id_type=pl.DeviceIdType.MESH,
            )
        pl.semaphore_wait(barrier, N_DEV - 1)

        cnt_rdmas = []
        for d in range(1, N_DEV):
            peer = lax.rem(my + d, N_DEV)
            r = pltpu.make_async_remote_copy(
                src_ref=cnt_ref,
                dst_ref=cnt_recv.at[d - 1],
                send_sem=csend.at[d - 1],
                recv_sem=crecv.at[d - 1],
                device_id=(peer,),
                device_id_type=pl.DeviceIdType.MESH,
            )
            r.start()
            cnt_rdmas.append(r)

        comm[0] = ew_ref[...]
        w_rdmas = []

        def start_hop(h):
            r = pltpu.make_async_remote_copy(
                src_ref=comm.at[h],
                dst_ref=comm.at[h + 1],
                send_sem=wsend.at[h],
                recv_sem=wrecv.at[h],
                device_id=(right,),
                device_id_type=pl.DeviceIdType.MESH,
            )
            r.start()
            w_rdmas.append(r)

        start_hop(0)

        for r in cnt_rdmas:
            r.wait()
        base = jnp.zeros((1, n_exp), jnp.float32)
        for d in range(1, N_DEV):
            base = base + jnp.where(my >= d, 1.0, 0.0) * cnt_recv[d - 1]
        route_v = route_ref[...]
        iota_e = lax.broadcasted_iota(jnp.int32, (1, n_exp), 1)
        oh = (route_v == iota_e).astype(jnp.float32)
        prior = (oh * base).sum(axis=1, keepdims=True) + excl_ref[...]
        keep = (prior < CAP).astype(jnp.float32)

        x_v = x_ref[...]

        def pair_out(p, w0, w1):
            e0 = e_local * p
            m0 = keep * (route_v == e0).astype(jnp.float32)
            m1 = keep * (route_v == e0 + 1).astype(jnp.float32)
            acc = jnp.dot(x_v * m0, w0, preferred_element_type=jnp.float32)
            return acc + jnp.dot(x_v * m1, w1, preferred_element_type=jnp.float32)

        out_ref[...] = pair_out(my, ew_ref[0], ew_ref[1])

        for h in range(N_DEV - 1):
            w_rdmas[h].wait()
            if h + 1 < N_DEV - 1:
                start_hop(h + 1)
            p = lax.rem(my - (h + 1) + N_DEV, N_DEV)
            out_ref[...] = out_ref[...] + pair_out(
                p, comm[h + 1, 0], comm[h + 1, 1]
            )

    return pl.pallas_call(
        body,
        out_shape=jax.ShapeDtypeStruct((n_tok, d_ff), jnp.float32),
        in_specs=[pl.BlockSpec(memory_space=pltpu.VMEM)] * 5,
        out_specs=pl.BlockSpec(memory_space=pltpu.VMEM),
        scratch_shapes=[
            pltpu.VMEM((N_DEV, e_local, d_model, d_ff), jnp.float32),
            pltpu.VMEM((N_DEV - 1, 1, n_exp), jnp.float32),
            pltpu.SemaphoreType.DMA((N_DEV - 1,)),
            pltpu.SemaphoreType.DMA((N_DEV - 1,)),
            pltpu.SemaphoreType.DMA((N_DEV - 1,)),
            pltpu.SemaphoreType.DMA((N_DEV - 1,)),
        ],
        compiler_params=pltpu.CompilerParams(collective_id=0),
    )(x, route, excl_tok, counts, expert_W)


# device time: 27177 ns/iter; 1.7424x vs baseline; 1.7424x over previous
import jax
import jax.numpy as jnp
from jax import lax
from jax.experimental import pallas as pl
from jax.experimental.pallas import tpu as pltpu

N_DEV = 4
CAP = 204.0


def kernel(x, router_W, route_idx, expert_W):
    n_tok, d_model = x.shape
    e_local, _, d_ff = expert_W.shape
    n_exp = N_DEV * e_local

    route = route_idx[:, :1].astype(jnp.int32)
    onehot = (route == jnp.arange(n_exp, dtype=jnp.int32)[None, :]).astype(
        jnp.float32
    )
    excl_tok = ((jnp.cumsum(onehot, axis=0) - onehot) * onehot).sum(
        axis=1, keepdims=True
    )
    counts = onehot.sum(axis=0, keepdims=True)

    def body(
        x_ref, route_ref, excl_ref, cnt_ref, ew_ref, out_ref,
        bufL, bufR, bufD, cnt_recv, wsend, wrecv, csend, crecv,
    ):
        my = lax.axis_index("i")
        right = lax.rem(my + 1, N_DEV)
        left = lax.rem(my + 3, N_DEV)
        diag = lax.rem(my + 2, N_DEV)

        barrier = pltpu.get_barrier_semaphore()
        for d in range(1, N_DEV):
            peer = lax.rem(my + d, N_DEV)
            pl.semaphore_signal(
                barrier, inc=1, device_id=(peer,),
                device_id_type=pl.DeviceIdType.MESH,
            )
        pl.semaphore_wait(barrier, N_DEV - 1)

        cnt_rdmas = []
        for d in range(1, N_DEV):
            peer = lax.rem(my + d, N_DEV)
            r = pltpu.make_async_remote_copy(
                src_ref=cnt_ref,
                dst_ref=cnt_recv.at[d - 1],
                send_sem=csend.at[d - 1],
                recv_sem=crecv.at[d - 1],
                device_id=(peer,),
                device_id_type=pl.DeviceIdType.MESH,
            )
            r.start()
            cnt_rdmas.append(r)

        def mk(slot, src, dst, tgt):
            return pltpu.make_async_remote_copy(
                src_ref=src, dst_ref=dst,
                send_sem=wsend.at[slot], recv_sem=wrecv.at[slot],
                device_id=(tgt,), device_id_type=pl.DeviceIdType.MESH,
            )

        r0 = mk(0, ew_ref.at[0], bufL.at[0], right)
        r1 = mk(1, ew_ref.at[1], bufL.at[1], right)
        r2 = mk(2, ew_ref.at[1], bufR.at[1], left)
        r3 = mk(3, ew_ref.at[0], bufR.at[0], left)
        for r in (r0, r1, r2, r3):
            r.start()

        for r in cnt_rdmas:
            r.wait()
        base = jnp.zeros((1, n_exp), jnp.float32)
        for d in range(1, N_DEV):
            base = base + jnp.where(my >= d, 1.0, 0.0) * cnt_recv[d - 1]
        route_v = route_ref[...]
        iota_e = lax.broadcasted_iota(jnp.int32, (1, n_exp), 1)
        oh = (route_v == iota_e).astype(jnp.float32)
        prior = (oh * base).sum(axis=1, keepdims=True) + excl_ref[...]
        keep = (prior < CAP).astype(jnp.float32)

        x_v = x_ref[...]

        def pair_out(p, w0, w1):
            e0 = e_local * p
            m0 = keep * (route_v == e0).astype(jnp.float32)
            m1 = keep * (route_v == e0 + 1).astype(jnp.float32)
            acc = jnp.dot(x_v * m0, w0, preferred_element_type=jnp.float32)
            return acc + jnp.dot(x_v * m1, w1, preferred_element_type=jnp.float32)

        out_ref[...] = pair_out(my, ew_ref[0], ew_ref[1])

        r0.wait()
        r4 = mk(4, bufL.at[0], bufD.at[0], right)
        r4.start()
        r2.wait()
        r5 = mk(5, bufR.at[1], bufD.at[1], left)
        r5.start()

        r1.wait()
        out_ref[...] = out_ref[...] + pair_out(left, bufL[0], bufL[1])
        r3.wait()
        out_ref[...] = out_ref[...] + pair_out(right, bufR[0], bufR[1])
        r4.wait()
        r5.wait()
        out_ref[...] = out_ref[...] + pair_out(diag, bufD[0], bufD[1])

    return pl.pallas_call(
        body,
        out_shape=jax.ShapeDtypeStruct((n_tok, d_ff), jnp.float32),
        in_specs=[pl.BlockSpec(memory_space=pltpu.VMEM)] * 5,
        out_specs=pl.BlockSpec(memory_space=pltpu.VMEM),
        scratch_shapes=[
            pltpu.VMEM((e_local, d_model, d_ff), jnp.float32),
            pltpu.VMEM((e_local, d_model, d_ff), jnp.float32),
            pltpu.VMEM((e_local, d_model, d_ff), jnp.float32),
            pltpu.VMEM((N_DEV - 1, 1, n_exp), jnp.float32),
            pltpu.SemaphoreType.DMA((6,)),
            pltpu.SemaphoreType.DMA((6,)),
            pltpu.SemaphoreType.DMA((N_DEV - 1,)),
            pltpu.SemaphoreType.DMA((N_DEV - 1,)),
        ],
        compiler_params=pltpu.CompilerParams(collective_id=0),
    )(x, route, excl_tok, counts, expert_W)


# device time: 18658 ns/iter; 2.5379x vs baseline; 1.4566x over previous
import jax
import jax.numpy as jnp
from jax import lax
from jax.experimental import pallas as pl
from jax.experimental.pallas import tpu as pltpu

N_DEV = 4
CAP = 204.0


def kernel(x, router_W, route_idx, expert_W):
    n_tok, d_model = x.shape
    e_local, _, d_ff = expert_W.shape
    n_exp = N_DEV * e_local

    route = route_idx[:, :1].astype(jnp.int32)
    onehot = (route == jnp.arange(n_exp, dtype=jnp.int32)[None, :]).astype(
        jnp.float32
    )
    excl_tok = ((jnp.cumsum(onehot, axis=0) - onehot) * onehot).sum(
        axis=1, keepdims=True
    )
    counts = onehot.sum(axis=0, keepdims=True)

    x_bf = x.astype(jnp.bfloat16)
    ew_bf = expert_W.astype(jnp.bfloat16)

    def body(
        x_ref, route_ref, excl_ref, cnt_ref, ew_ref, out_ref,
        bufL, bufR, bufD, cnt_recv, wsend, wrecv, csend, crecv,
    ):
        my = lax.axis_index("i")
        right = lax.rem(my + 1, N_DEV)
        left = lax.rem(my + 3, N_DEV)
        diag = lax.rem(my + 2, N_DEV)

        barrier = pltpu.get_barrier_semaphore()
        for d in range(1, N_DEV):
            peer = lax.rem(my + d, N_DEV)
            pl.semaphore_signal(
                barrier, inc=1, device_id=(peer,),
                device_id_type=pl.DeviceIdType.MESH,
            )
        pl.semaphore_wait(barrier, N_DEV - 1)

        cnt_rdmas = []
        for d in range(1, N_DEV):
            peer = lax.rem(my + d, N_DEV)
            r = pltpu.make_async_remote_copy(
                src_ref=cnt_ref,
                dst_ref=cnt_recv.at[d - 1],
                send_sem=csend.at[d - 1],
                recv_sem=crecv.at[d - 1],
                device_id=(peer,),
                device_id_type=pl.DeviceIdType.MESH,
            )
            r.start()
            cnt_rdmas.append(r)

        def mk(slot, src, dst, tgt):
            return pltpu.make_async_remote_copy(
                src_ref=src, dst_ref=dst,
                send_sem=wsend.at[slot], recv_sem=wrecv.at[slot],
                device_id=(tgt,), device_id_type=pl.DeviceIdType.MESH,
            )

        r0 = mk(0, ew_ref.at[0], bufL.at[0], right)
        r1 = mk(1, ew_ref.at[1], bufL.at[1], right)
        r2 = mk(2, ew_ref.at[1], bufR.at[1], left)
        r3 = mk(3, ew_ref.at[0], bufR.at[0], left)
        for r in (r0, r1, r2, r3):
            r.start()

        for r in cnt_rdmas:
            r.wait()
        base = jnp.zeros((1, n_exp), jnp.float32)
        for d in range(1, N_DEV):
            base = base + jnp.where(my >= d, 1.0, 0.0) * cnt_recv[d - 1]
        route_v = route_ref[...]
        iota_e = lax.broadcasted_iota(jnp.int32, (1, n_exp), 1)
        oh = (route_v == iota_e).astype(jnp.float32)
        prior = (oh * base).sum(axis=1, keepdims=True) + excl_ref[...]
        keep = (prior < CAP).astype(jnp.float32)

        x_v = x_ref[...]

        def pair_out(p, w0, w1):
            e0 = e_local * p
            m0 = (keep * (route_v == e0).astype(jnp.float32)).astype(jnp.bfloat16)
            m1 = (keep * (route_v == e0 + 1).astype(jnp.float32)).astype(jnp.bfloat16)
            acc = jnp.dot(x_v * m0, w0, preferred_element_type=jnp.float32)
            return acc + jnp.dot(x_v * m1, w1, preferred_element_type=jnp.float32)

        out_ref[...] = pair_out(my, ew_ref[0], ew_ref[1])

        r0.wait()
        r4 = mk(4, bufL.at[0], bufD.at[0], right)
        r4.start()
        r2.wait()
        r5 = mk(5, bufR.at[1], bufD.at[1], left)
        r5.start()

        r1.wait()
        out_ref[...] = out_ref[...] + pair_out(left, bufL[0], bufL[1])
        r3.wait()
        out_ref[...] = out_ref[...] + pair_out(right, bufR[0], bufR[1])
        r4.wait()
        r5.wait()
        out_ref[...] = out_ref[...] + pair_out(diag, bufD[0], bufD[1])

    return pl.pallas_call(
        body,
        out_shape=jax.ShapeDtypeStruct((n_tok, d_ff), jnp.float32),
        in_specs=[pl.BlockSpec(memory_space=pltpu.VMEM)] * 5,
        out_specs=pl.BlockSpec(memory_space=pltpu.VMEM),
        scratch_shapes=[
            pltpu.VMEM((e_local, d_model, d_ff), jnp.bfloat16),
            pltpu.VMEM((e_local, d_model, d_ff), jnp.bfloat16),
            pltpu.VMEM((e_local, d_model, d_ff), jnp.bfloat16),
            pltpu.VMEM((N_DEV - 1, 1, n_exp), jnp.float32),
            pltpu.SemaphoreType.DMA((6,)),
            pltpu.SemaphoreType.DMA((6,)),
            pltpu.SemaphoreType.DMA((N_DEV - 1,)),
            pltpu.SemaphoreType.DMA((N_DEV - 1,)),
        ],
        compiler_params=pltpu.CompilerParams(collective_id=0),
    )(x_bf, route, excl_tok, counts, ew_bf)
